# Initial kernel scaffold; baseline (speedup 1.0000x reference)
import jax
import jax.numpy as jnp
from jax.experimental import pallas as pl


def kernel(t, intervals):
    raise NotImplementedError("write your pallas kernel here")


# speedup vs baseline: 1.1434x; 1.1434x over previous
import functools

import jax
import jax.numpy as jnp
from jax import lax
from jax.experimental import pallas as pl
from jax.experimental.pallas import tpu as pltpu
from jax.experimental.pallas import tpu_sc as plsc

_K = 128
_L = 16
_CHUNKS = _K // _L


def _sc_body(t_hbm, lo_hbm, hi_hbm, out_hbm, t_v, lo_v, hi_v, out_v):
    cid = lax.axis_index("c")
    sid = lax.axis_index("s")

    @pl.when(jnp.logical_and(cid == 0, sid == 0))
    def _():
        pltpu.sync_copy(t_hbm, t_v)
        pltpu.sync_copy(lo_hbm, lo_v)
        pltpu.sync_copy(hi_hbm, hi_v)

        tv = t_v[...]
        best = jnp.full((_L,), _K, dtype=jnp.int32)
        for i in range(_CHUNKS):
            lov = lo_v[pl.ds(i * _L, _L)]
            hiv = hi_v[pl.ds(i * _L, _L)]
            cond = jnp.logical_and(tv >= lov, tv <= hiv)
            ffs = plsc.all_reduce_ffs(cond)
            cand = jnp.where(ffs >= _L, _K, ffs + (i * _L))
            best = jnp.minimum(best, cand)

        res = jnp.where(best >= _K, jnp.full((_L,), -1, jnp.int32), best)
        out_v[...] = res
        pltpu.sync_copy(out_v, out_hbm)


@jax.jit
def _time_to_interval(tv, lo, hi):
    run = functools.partial(
        pl.kernel,
        out_type=jax.ShapeDtypeStruct((_L,), jnp.int32),
        mesh=plsc.VectorSubcoreMesh(core_axis_name="c", subcore_axis_name="s"),
        compiler_params=pltpu.CompilerParams(needs_layout_passes=False),
        scratch_types=[
            pltpu.VMEM((_L,), jnp.float32),
            pltpu.VMEM((_K,), jnp.float32),
            pltpu.VMEM((_K,), jnp.float32),
            pltpu.VMEM((_L,), jnp.int32),
        ],
    )(_sc_body)
    return run(tv, lo, hi)


def kernel(t, intervals):
    tv = jnp.broadcast_to(jnp.asarray(t, jnp.float32), (_L,))
    lo = jnp.asarray(intervals[:, 0], jnp.float32)
    hi = jnp.asarray(intervals[:, 1], jnp.float32)
    out = _time_to_interval(tv, lo, hi)
    return out[0]


# speedup vs baseline: 1.1718x; 1.0249x over previous
import functools

import jax
import jax.numpy as jnp
from jax import lax
from jax.experimental import pallas as pl
from jax.experimental.pallas import tpu as pltpu
from jax.experimental.pallas import tpu_sc as plsc

_K = 128
_L = 16
_CHUNKS = _K // _L


def _sc_body(t_hbm, flat_hbm, out_hbm, t_v, flat_v, out_v):
    cid = lax.axis_index("c")
    sid = lax.axis_index("s")

    @pl.when(jnp.logical_and(cid == 0, sid == 0))
    def _():
        pltpu.sync_copy(t_hbm, t_v.at[pl.ds(0, 1)])
        pltpu.sync_copy(flat_hbm, flat_v)

        zero = jnp.zeros((_L,), jnp.int32)
        tv = plsc.load_gather(t_v, [zero])
        even = 2 * lax.broadcasted_iota(jnp.int32, (_L,), 0)
        best = jnp.full((_L,), _K, dtype=jnp.int32)
        for i in range(_CHUNKS):
            lov = plsc.load_gather(flat_v, [even + 2 * _L * i])
            hiv = plsc.load_gather(flat_v, [even + (2 * _L * i + 1)])
            cond = jnp.logical_and(tv >= lov, tv <= hiv)
            ffs = plsc.all_reduce_ffs(cond)
            cand = jnp.where(ffs >= _L, _K, ffs + (i * _L))
            best = jnp.minimum(best, cand)

        res = jnp.where(best >= _K, jnp.full((_L,), -1, jnp.int32), best)
        out_v[...] = res
        pltpu.sync_copy(out_v.at[pl.ds(0, 1)], out_hbm)


@jax.jit
def _time_to_interval(tv, flat):
    run = functools.partial(
        pl.kernel,
        out_type=jax.ShapeDtypeStruct((1,), jnp.int32),
        mesh=plsc.VectorSubcoreMesh(core_axis_name="c", subcore_axis_name="s"),
        compiler_params=pltpu.CompilerParams(needs_layout_passes=False),
        scratch_types=[
            pltpu.VMEM((_L,), jnp.float32),
            pltpu.VMEM((2 * _K,), jnp.float32),
            pltpu.VMEM((_L,), jnp.int32),
        ],
    )(_sc_body)
    return run(tv, flat)


def kernel(t, intervals):
    tv = jnp.reshape(jnp.asarray(t, jnp.float32), (1,))
    flat = jnp.reshape(jnp.asarray(intervals, jnp.float32), (2 * _K,))
    out = _time_to_interval(tv, flat)
    return jnp.reshape(out, ())


# speedup vs baseline: 1.2616x; 1.0766x over previous
import functools

import jax
import jax.numpy as jnp
from jax import lax
from jax.experimental import pallas as pl
from jax.experimental.pallas import tpu as pltpu
from jax.experimental.pallas import tpu_sc as plsc

_K = 128
_L = 16
_CHUNKS = _K // _L


def _sc_body(t_hbm, flat_hbm, out_hbm, t_v, flat_v, out_v):
    cid = lax.axis_index("c")
    sid = lax.axis_index("s")

    @pl.when(jnp.logical_and(cid == 0, sid == 0))
    def _():
        pltpu.sync_copy(t_hbm, t_v.at[pl.ds(0, 1)])
        pltpu.sync_copy(flat_hbm, flat_v)

        zero = jnp.zeros((_L,), jnp.int32)
        tv = plsc.load_gather(t_v, [zero])
        even = 2 * lax.broadcasted_iota(jnp.int32, (_L,), 0)
        best = jnp.full((_L,), _K, dtype=jnp.int32)
        for i in range(_CHUNKS):
            lov = plsc.load_gather(flat_v, [even + 2 * _L * i])
            hiv = plsc.load_gather(flat_v, [even + (2 * _L * i + 1)])
            cond = jnp.logical_and(tv >= lov, tv <= hiv)
            ffs = plsc.all_reduce_ffs(cond)
            cand = jnp.where(ffs >= _L, _K, ffs + (i * _L))
            best = jnp.minimum(best, cand)

        res = jnp.where(best >= _K, jnp.full((_L,), -1, jnp.int32), best)
        out_v[...] = res
        pltpu.sync_copy(out_v.at[pl.ds(0, 1)], out_hbm)


@jax.jit
def _time_to_interval(tv, flat):
    run = functools.partial(
        pl.kernel,
        out_type=jax.ShapeDtypeStruct((1,), jnp.int32),
        mesh=plsc.VectorSubcoreMesh(
            core_axis_name="c", subcore_axis_name="s", num_cores=1
        ),
        compiler_params=pltpu.CompilerParams(needs_layout_passes=False),
        scratch_types=[
            pltpu.VMEM((_L,), jnp.float32),
            pltpu.VMEM((2 * _K,), jnp.float32),
            pltpu.VMEM((_L,), jnp.int32),
        ],
    )(_sc_body)
    return run(tv, flat)


def kernel(t, intervals):
    tv = jnp.reshape(jnp.asarray(t, jnp.float32), (1,))
    flat = jnp.reshape(jnp.asarray(intervals, jnp.float32), (2 * _K,))
    out = _time_to_interval(tv, flat)
    return jnp.reshape(out, ())


# speedup vs baseline: 1.2647x; 1.0024x over previous
import functools

import jax
import jax.numpy as jnp
from jax import lax
from jax.experimental import pallas as pl
from jax.experimental.pallas import tpu as pltpu
from jax.experimental.pallas import tpu_sc as plsc

_K = 128
_L = 16
_CHUNKS = _K // _L


def _sc_body(t_hbm, flat_hbm, out_hbm, t_v, flat_v, out_v):
    cid = lax.axis_index("c")
    sid = lax.axis_index("s")

    @pl.when(jnp.logical_and(cid == 0, sid == 0))
    def _():
        pltpu.sync_copy(t_hbm, t_v.at[pl.ds(0, 1)])
        pltpu.sync_copy(flat_hbm, flat_v)

        zero = jnp.zeros((_L,), jnp.int32)
        tv = plsc.load_gather(t_v, [zero])
        even = 2 * lax.broadcasted_iota(jnp.int32, (_L,), 0)
        best = jnp.full((_L,), _K, dtype=jnp.int32)
        for i in range(_CHUNKS):
            lov = plsc.load_gather(flat_v, [even + 2 * _L * i])
            hiv = plsc.load_gather(flat_v, [even + (2 * _L * i + 1)])
            cond = jnp.logical_and(tv >= lov, tv <= hiv)
            ffs = plsc.all_reduce_ffs(cond)
            cand = jnp.where(ffs >= _L, _K, ffs + (i * _L))
            best = jnp.minimum(best, cand)

        res = jnp.where(best >= _K, jnp.full((_L,), -1, jnp.int32), best)
        out_v[...] = res
        pltpu.sync_copy(out_v.at[pl.ds(0, 1)], out_hbm)


@jax.jit
def _time_to_interval(tv, flat):
    run = functools.partial(
        pl.kernel,
        out_type=jax.ShapeDtypeStruct((1,), jnp.int32),
        mesh=plsc.VectorSubcoreMesh(
            core_axis_name="c", subcore_axis_name="s", num_cores=1,
            num_subcores=1
        ),
        compiler_params=pltpu.CompilerParams(needs_layout_passes=False),
        scratch_types=[
            pltpu.VMEM((_L,), jnp.float32),
            pltpu.VMEM((2 * _K,), jnp.float32),
            pltpu.VMEM((_L,), jnp.int32),
        ],
    )(_sc_body)
    return run(tv, flat)


def kernel(t, intervals):
    tv = jnp.reshape(jnp.asarray(t, jnp.float32), (1,))
    flat = jnp.reshape(jnp.asarray(intervals, jnp.float32), (2 * _K,))
    out = _time_to_interval(tv, flat)
    return jnp.reshape(out, ())
